# Initial kernel scaffold; baseline (speedup 1.0000x reference)
import jax
import jax.numpy as jnp
from jax.experimental import pallas as pl


def kernel(input, table):
    raise NotImplementedError("write your pallas kernel here")


# speedup vs baseline: 4.8069x; 4.8069x over previous
import functools

import jax
import jax.numpy as jnp
from jax import lax
from jax.experimental import pallas as pl
from jax.experimental.pallas import tpu as pltpu
from jax.experimental.pallas import tpu_sc as plsc

D = 32
NC = 2
NS = 16
NW = NC * NS
CHUNK = 1024


@functools.lru_cache(maxsize=None)
def _gather_fn(N: int):
    b_per_w = N // NW
    n_chunks = b_per_w // CHUNK
    mesh = plsc.VectorSubcoreMesh(core_axis_name="c", subcore_axis_name="s")

    @functools.partial(
        pl.kernel,
        mesh=mesh,
        out_type=jax.ShapeDtypeStruct((N, D), jnp.float32),
        scratch_types=[
            pltpu.VMEM((CHUNK,), jnp.int32),
            pltpu.VMEM((CHUNK, D), jnp.float32),
            pltpu.SemaphoreType.DMA,
        ],
        compiler_params=pltpu.CompilerParams(use_tc_tiling_on_sc=False),
    )
    def gather(idx_hbm, table_hbm, out_hbm, idx_v, rows_v, sem):
        wid = lax.axis_index("s") * NC + lax.axis_index("c")
        base = wid * b_per_w

        def body(j, carry):
            off = base + j * CHUNK
            pltpu.sync_copy(idx_hbm.at[pl.ds(off, CHUNK)], idx_v)
            pltpu.async_copy(table_hbm.at[idx_v], rows_v, sem).wait()
            pltpu.sync_copy(rows_v, out_hbm.at[pl.ds(off, CHUNK)])
            return carry

        lax.fori_loop(0, n_chunks, body, 0)

    return gather


def kernel(input, table):
    B, H = input.shape
    N = B * H
    idx = input.reshape(N)
    out = _gather_fn(N)(idx, table)
    return out.reshape(B, H, D)


# speedup vs baseline: 5.0358x; 1.0476x over previous
import functools

import jax
import jax.numpy as jnp
from jax import lax
from jax.experimental import pallas as pl
from jax.experimental.pallas import tpu as pltpu
from jax.experimental.pallas import tpu_sc as plsc

D = 32
NC = 2
NS = 16
NW = NC * NS
CHUNK = 1600
CPB = 8
IBLK = CHUNK * CPB


@functools.lru_cache(maxsize=None)
def _gather_fn(N: int):
    b_per_w = N // NW
    n_chunks = b_per_w // CHUNK
    n_blocks = n_chunks // CPB
    assert n_blocks % 2 == 0 and n_chunks == n_blocks * CPB
    mesh = plsc.VectorSubcoreMesh(core_axis_name="c", subcore_axis_name="s")

    @functools.partial(
        pl.kernel,
        mesh=mesh,
        out_type=jax.ShapeDtypeStruct((N, D), jnp.float32),
        scratch_types=[
            pltpu.VMEM((IBLK,), jnp.int32),
            pltpu.VMEM((IBLK,), jnp.int32),
            pltpu.VMEM((2, CHUNK, D), jnp.float32),
            pltpu.SemaphoreType.DMA((2,)),
            pltpu.SemaphoreType.DMA,
            pltpu.SemaphoreType.DMA((2,)),
        ],
        compiler_params=pltpu.CompilerParams(use_tc_tiling_on_sc=False),
    )
    def gather(idx_hbm, table_hbm, out_hbm, idx_v0, idx_v1, rows_v,
               isem, gsem, osem):
        wid = lax.axis_index("s") * NC + lax.axis_index("c")
        base = wid * b_per_w
        idx_bufs = (idx_v0, idx_v1)

        def idx_load(kb, slot):
            return pltpu.make_async_copy(
                idx_hbm.at[pl.ds(base + kb * IBLK, IBLK)],
                idx_bufs[slot], isem.at[slot])

        idx_load(0, 0).start()
        idx_load(1, 1).start()

        def out_copy(c, b):
            return pltpu.make_async_copy(
                rows_v.at[b],
                out_hbm.at[pl.ds(base + c * CHUNK, CHUNK)],
                osem.at[b])

        def chunk_steps(kb, slot):
            def r_body(r, carry):
                c = kb * CPB + r
                b = r % 2

                @pl.when(r == 0)
                def _():
                    idx_load(kb, slot).wait()

                @pl.when(c >= 2)
                def _():
                    out_copy(c - 2, b).wait()

                pltpu.async_copy(
                    table_hbm.at[idx_bufs[slot].at[pl.ds(r * CHUNK, CHUNK)]],
                    rows_v.at[b], gsem).wait()

                out_copy(c, b).start()

                @pl.when((r == CPB - 1) & (kb + 2 < n_blocks))
                def _():
                    idx_load(kb + 2, slot).start()

                return carry

            lax.fori_loop(0, CPB, r_body, 0)

        def pair_body(i, carry):
            chunk_steps(2 * i, 0)
            chunk_steps(2 * i + 1, 1)
            return carry

        lax.fori_loop(0, n_blocks // 2, pair_body, 0)

        for b in (0, 1):
            out_copy(n_chunks - 2 + b, b).wait()

    return gather


def kernel(input, table):
    B, H = input.shape
    N = B * H
    idx = input.reshape(N)
    out = _gather_fn(N)(idx, table)
    return out.reshape(B, H, D)


# speedup vs baseline: 5.0510x; 1.0030x over previous
import functools

import jax
import jax.numpy as jnp
from jax import lax
from jax.experimental import pallas as pl
from jax.experimental.pallas import tpu as pltpu
from jax.experimental.pallas import tpu_sc as plsc

D = 32
NC = 2
NS = 16
NW = NC * NS
CHUNK = 800
NBUF = 4
LAG = 2
CPB = 8
IBLK = CHUNK * CPB


@functools.lru_cache(maxsize=None)
def _gather_fn(N: int):
    b_per_w = N // NW
    n_chunks = b_per_w // CHUNK
    n_blocks = n_chunks // CPB
    assert n_chunks == n_blocks * CPB and n_blocks % 2 == 0
    assert n_chunks % NBUF == 0 and CPB % NBUF == 0
    mesh = plsc.VectorSubcoreMesh(core_axis_name="c", subcore_axis_name="s")

    @functools.partial(
        pl.kernel,
        mesh=mesh,
        out_type=jax.ShapeDtypeStruct((N, D), jnp.float32),
        scratch_types=[
            pltpu.VMEM((IBLK,), jnp.int32),
            pltpu.VMEM((IBLK,), jnp.int32),
            pltpu.VMEM((NBUF, CHUNK, D), jnp.float32),
            pltpu.SemaphoreType.DMA((2,)),
            pltpu.SemaphoreType.DMA((NBUF,)),
            pltpu.SemaphoreType.DMA((NBUF,)),
        ],
        compiler_params=pltpu.CompilerParams(use_tc_tiling_on_sc=False),
    )
    def gather(idx_hbm, table_hbm, out_hbm, idx_v0, idx_v1, rows_v,
               isem, gsem, osem):
        wid = lax.axis_index("s") * NC + lax.axis_index("c")
        base = wid * b_per_w
        idx_bufs = (idx_v0, idx_v1)

        def idx_load(kb, slot):
            return pltpu.make_async_copy(
                idx_hbm.at[pl.ds(base + kb * IBLK, IBLK)],
                idx_bufs[slot], isem.at[slot])

        def gather_copy(idx_ref, r, b):
            return pltpu.make_async_copy(
                table_hbm.at[idx_ref.at[pl.ds(r * CHUNK, CHUNK)]],
                rows_v.at[b], gsem.at[b])

        def out_copy(c, b):
            return pltpu.make_async_copy(
                rows_v.at[b],
                out_hbm.at[pl.ds(base + c * CHUNK, CHUNK)],
                osem.at[b])

        idx_load(0, 0).start()

        def chunk_steps(kb, slot):
            def r_body(r, carry):
                c = kb * CPB + r
                b = r % NBUF

                @pl.when(r == 0)
                def _():
                    idx_load(kb, slot).wait()

                @pl.when(c >= NBUF)
                def _():
                    out_copy(c - NBUF, b).wait()

                gather_copy(idx_bufs[slot], r, b).start()

                @pl.when((r == LAG) & (kb + 1 < n_blocks))
                def _():
                    idx_load(kb + 1, 1 - slot).start()

                @pl.when(c >= LAG)
                def _():
                    cp = c - LAG
                    bp = cp % NBUF
                    gather_copy(idx_bufs[slot], r, bp).wait()
                    out_copy(cp, bp).start()

                return carry

            lax.fori_loop(0, CPB, r_body, 0)

        def pair_body(i, carry):
            chunk_steps(2 * i, 0)
            chunk_steps(2 * i + 1, 1)
            return carry

        lax.fori_loop(0, n_blocks // 2, pair_body, 0)

        for t in range(LAG):
            c = n_chunks - LAG + t
            b = c % NBUF
            gather_copy(idx_bufs[(n_blocks - 1) % 2], 0, b).wait()
            out_copy(c, b).start()
        for t in range(NBUF):
            c = n_chunks - NBUF + t
            out_copy(c, c % NBUF).wait()

    return gather


def kernel(input, table):
    B, H = input.shape
    N = B * H
    idx = input.reshape(N)
    out = _gather_fn(N)(idx, table)
    return out.reshape(B, H, D)
